# baseline (speedup 1.0000x reference)
import functools

import jax
import jax.numpy as jnp
from jax import lax
from jax.experimental import pallas as pl
from jax.experimental.pallas import tpu as pltpu

_INTERPRET = False

_E, _K, _G, _TG = 8, 2, 4, 2


def _router_shared_body(x_ref, wsg_ref, wsu_ref, wsd_ref, wr_ref, eb_ref,
                        shared_ref, e0_ref, e1_ref, w0_ref, w1_ref):
    xb = x_ref[...]
    g = jnp.dot(xb, wsg_ref[...], preferred_element_type=jnp.float32)
    u = jnp.dot(xb, wsu_ref[...], preferred_element_type=jnp.float32)
    a = g * jax.nn.sigmoid(g) * u
    shared_ref[...] = jnp.dot(a, wsd_ref[...], preferred_element_type=jnp.float32)

    logits = jnp.dot(xb, wr_ref[...], preferred_element_type=jnp.float32)
    scores = jax.nn.sigmoid(logits)
    eb = eb_ref[...]
    cols = [scores[:, e:e + 1] for e in range(_E)]
    colsb = [cols[e] + eb[:, e:e + 1] for e in range(_E)]

    epg = _E // _G
    gs = []
    for gi in range(_G):
        m = colsb[gi * epg]
        for j in range(1, epg):
            m = jnp.maximum(m, colsb[gi * epg + j])
        gs.append(m)
    keep_g = []
    for gi in range(_G):
        rank = jnp.zeros_like(gs[0], dtype=jnp.int32)
        for gj in range(_G):
            if gj == gi:
                continue
            beats = (gs[gj] > gs[gi]) if gj > gi else (gs[gj] >= gs[gi])
            rank += beats.astype(jnp.int32)
        keep_g.append(rank < _TG)
    NEG = jnp.float32(-1e30)
    ms = [jnp.where(keep_g[e // epg], colsb[e], NEG) for e in range(_E)]
    sel = []
    for e in range(_E):
        rank = jnp.zeros_like(ms[0], dtype=jnp.int32)
        for e2 in range(_E):
            if e2 == e:
                continue
            beats = (ms[e2] > ms[e]) if e2 > e else (ms[e2] >= ms[e])
            rank += beats.astype(jnp.int32)
        sel.append(rank < _K)
    big = jnp.full_like(rank, 127)
    e0 = big
    e1 = jnp.full_like(rank, -1)
    for e in range(_E):
        e0 = jnp.where(sel[e], jnp.minimum(e0, e), e0)
        e1 = jnp.where(sel[e], jnp.maximum(e1, e), e1)
    w0 = jnp.zeros_like(cols[0])
    w1 = jnp.zeros_like(cols[0])
    for e in range(_E):
        w0 = jnp.where(sel[e] & (e0 == e), cols[e], w0)
        w1 = jnp.where(sel[e] & (e1 == e), cols[e], w1)
    denom = w0 + w1 + jnp.float32(1e-20)
    e0_ref[...] = e0
    e1_ref[...] = e1
    w0_ref[...] = w0 / denom
    w1_ref[...] = w1 / denom


def _router_shared(xf, wr_t, wsg_t, wsu_t, wsd_t, eb):
    S, C = xf.shape
    HS = wsg_t.shape[1]
    BM = 256
    grid = (S // BM,)
    return pl.pallas_call(
        _router_shared_body,
        grid=grid,
        in_specs=[
            pl.BlockSpec((BM, C), lambda i: (i, 0)),
            pl.BlockSpec((C, HS), lambda i: (0, 0)),
            pl.BlockSpec((C, HS), lambda i: (0, 0)),
            pl.BlockSpec((HS, C), lambda i: (0, 0)),
            pl.BlockSpec((C, _E), lambda i: (0, 0)),
            pl.BlockSpec((1, _E), lambda i: (0, 0)),
        ],
        out_specs=[
            pl.BlockSpec((BM, C), lambda i: (i, 0)),
            pl.BlockSpec((BM, 1), lambda i: (i, 0)),
            pl.BlockSpec((BM, 1), lambda i: (i, 0)),
            pl.BlockSpec((BM, 1), lambda i: (i, 0)),
            pl.BlockSpec((BM, 1), lambda i: (i, 0)),
        ],
        out_shape=[
            jax.ShapeDtypeStruct((S, C), jnp.float32),
            jax.ShapeDtypeStruct((S, 1), jnp.int32),
            jax.ShapeDtypeStruct((S, 1), jnp.int32),
            jax.ShapeDtypeStruct((S, 1), jnp.float32),
            jax.ShapeDtypeStruct((S, 1), jnp.float32),
        ],
        interpret=_INTERPRET,
    )(xf, wsg_t, wsu_t, wsd_t, wr_t, eb)


_BM = 128


def _grouped_gemm_body(bexp_ref, xg_ref, wg_ref, wu_ref, wd_ref, ws_ref,
                       out_ref):
    xb = xg_ref[...]
    g = jnp.dot(xb, wg_ref[0], preferred_element_type=jnp.float32)
    u = jnp.dot(xb, wu_ref[0], preferred_element_type=jnp.float32)
    a = g * jax.nn.sigmoid(g) * u
    y = jnp.dot(a, wd_ref[0], preferred_element_type=jnp.float32)
    out_ref[...] = y * ws_ref[...]


def _grouped_gemm(xg, wg_t, wu_t, wd_t, w_sorted, blockexp):
    PT, C = xg.shape
    E, _, H = wg_t.shape
    NB = PT // _BM
    grid_spec = pltpu.PrefetchScalarGridSpec(
        num_scalar_prefetch=1,
        grid=(NB,),
        in_specs=[
            pl.BlockSpec((_BM, C), lambda b, be: (b, 0)),
            pl.BlockSpec((1, C, H), lambda b, be: (be[b], 0, 0)),
            pl.BlockSpec((1, C, H), lambda b, be: (be[b], 0, 0)),
            pl.BlockSpec((1, H, C), lambda b, be: (be[b], 0, 0)),
            pl.BlockSpec((_BM, 1), lambda b, be: (b, 0)),
        ],
        out_specs=pl.BlockSpec((_BM, C), lambda b, be: (b, 0)),
    )
    return pl.pallas_call(
        _grouped_gemm_body,
        grid_spec=grid_spec,
        out_shape=jax.ShapeDtypeStruct((PT, C), jnp.float32),
        interpret=_INTERPRET,
    )(blockexp, xg, wg_t, wu_t, wd_t, w_sorted)


def kernel(x, Wr, Wg, Wu, Wd, Wsg, Wsu, Wsd, e_bias):
    B, T, C = x.shape
    S = B * T
    E, H, _ = Wg.shape
    xf = x.reshape(S, C)

    shared, e0, e1, w0, w1 = _router_shared(
        xf, Wr.T, Wsg.T, Wsu.T, Wsd.T, e_bias.reshape(1, E))

    NB = (S * _K) // _BM + E
    PT = NB * _BM
    eflat = jnp.stack([e0[:, 0], e1[:, 0]], axis=1).reshape(S * _K)
    wflat = jnp.stack([w0[:, 0], w1[:, 0]], axis=1).reshape(S * _K)
    sort_idx = jnp.argsort(eflat, stable=True)
    es = eflat[sort_idx]
    cnt = jnp.bincount(eflat, length=E)
    padded = ((cnt + _BM - 1) // _BM) * _BM
    base = jnp.concatenate([jnp.zeros(1, jnp.int32),
                            jnp.cumsum(padded).astype(jnp.int32)])
    seg_start = jnp.concatenate([jnp.zeros(1, jnp.int32),
                                 jnp.cumsum(cnt).astype(jnp.int32)])
    rank = jnp.arange(S * _K, dtype=jnp.int32) - seg_start[es]
    dest_sorted = base[es] + rank
    src_token = jnp.zeros(PT, jnp.int32).at[dest_sorted].set(
        (sort_idx // _K).astype(jnp.int32))
    w_sorted = jnp.zeros(PT, jnp.float32).at[dest_sorted].set(wflat[sort_idx])
    dest_flat = jnp.zeros(S * _K, jnp.int32).at[sort_idx].set(dest_sorted)
    blockexp = jnp.minimum(
        jnp.searchsorted(base[1:], jnp.arange(NB, dtype=jnp.int32) * _BM,
                         side='right'),
        E - 1).astype(jnp.int32)

    xg = xf[src_token]
    yw = _grouped_gemm(xg, Wg.transpose(0, 2, 1), Wu.transpose(0, 2, 1),
                       Wd.transpose(0, 2, 1), w_sorted.reshape(PT, 1),
                       blockexp)
    dest2 = dest_flat.reshape(S, _K)
    routed = yw[dest2[:, 0]] + yw[dest2[:, 1]]
    return (shared + routed).reshape(B, T, C)
